# baseline (speedup 1.0000x reference)
import jax
import jax.numpy as jnp
from jax.experimental import pallas as pl
from jax.experimental.pallas import tpu as pltpu

_GRID = 32
_XBLK = 262144 // _GRID
_ZBLK = 16384 // _GRID
_XHEAD = 16
_ZHEAD = 8


def _fused_kernel(x_ref, y_ref, z_ref, w_ref, xo_ref, zo_ref):
    i = pl.program_id(0)

    xo_ref[...] = x_ref[...]
    zo_ref[...] = z_ref[...]

    @pl.when(i == 0)
    def _fixup():
        xb = x_ref[0:_XHEAD, :]
        rows = jax.lax.broadcasted_iota(jnp.int32, xb.shape, 0)
        xb = jnp.where(rows == 10, y_ref[0:1, :], xb)
        xb = jnp.where(rows == 2, y_ref[1:2, :], xb)
        xb = jnp.where(rows == 1, jnp.float32(45.0), xb)
        xo_ref[0:_XHEAD, :] = xb

        zb = z_ref[0:_ZHEAD, :]
        rows = jax.lax.broadcasted_iota(jnp.int32, zb.shape, 0)
        cols = jax.lax.broadcasted_iota(jnp.int32, zb.shape, 1)
        upd = jnp.where((rows == 1) & (cols == 3), w_ref[0], 0.0)
        upd = jnp.where((rows == 0) & (cols == 2), w_ref[1], upd)
        upd = jnp.where((rows == 0) & (cols == 1), w_ref[2], upd)
        zo_ref[0:_ZHEAD, :] = zb + upd


def kernel(x, y, z, w):
    return pl.pallas_call(
        _fused_kernel,
        grid=(_GRID,),
        in_specs=[
            pl.BlockSpec((_XBLK, x.shape[1]), lambda i: (i, 0)),
            pl.BlockSpec((2, x.shape[1]), lambda i: (0, 0)),
            pl.BlockSpec((_ZBLK, z.shape[1]), lambda i: (i, 0)),
            pl.BlockSpec(memory_space=pltpu.SMEM),
        ],
        out_specs=[
            pl.BlockSpec((_XBLK, x.shape[1]), lambda i: (i, 0)),
            pl.BlockSpec((_ZBLK, z.shape[1]), lambda i: (i, 0)),
        ],
        out_shape=[
            jax.ShapeDtypeStruct(x.shape, x.dtype),
            jax.ShapeDtypeStruct(z.shape, z.dtype),
        ],
        compiler_params=pltpu.CompilerParams(dimension_semantics=("parallel",)),
    )(x, y, z, w)
